# baseline (speedup 1.0000x reference)
import functools

import jax
import jax.numpy as jnp
from jax import lax
from jax.experimental import pallas as pl
from jax.experimental.pallas import tpu as pltpu
from jax.experimental.pallas import tpu_sc as plsc

_R = 512


@functools.lru_cache(maxsize=None)
def _make_relayout(V, D):
    grid = -(-V // (2 * _R))
    H2 = grid * _R

    def relayout_body(lo_ref, hi_ref, eye_ref, out_ref):
        t0 = lax.dot_general(
            lo_ref[...], eye_ref[...], (((0,), (0,)), ((), ())),
            preferred_element_type=jnp.float32,
            precision=lax.Precision.HIGHEST,
        )
        t1 = lax.dot_general(
            hi_ref[...], eye_ref[...], (((0,), (0,)), ((), ())),
            preferred_element_type=jnp.float32,
            precision=lax.Precision.HIGHEST,
        )
        out_ref[:, 0:D] = t0
        out_ref[:, D : 2 * D] = t1

    return pl.pallas_call(
        relayout_body,
        grid=(grid,),
        in_specs=[
            pl.BlockSpec((D, _R), lambda p: (0, 2 * p)),
            pl.BlockSpec((D, _R), lambda p: (0, 2 * p + 1)),
            pl.BlockSpec((D, D), lambda p: (0, 0)),
        ],
        out_specs=pl.BlockSpec((_R, 2 * D), lambda p: (p, 0)),
        out_shape=jax.ShapeDtypeStruct((H2, 2 * D), jnp.float32),
    )


@functools.lru_cache(maxsize=None)
def _make_gather(V, D, B, H2):
    info = plsc.get_sparse_core_info()
    NC, NS = info.num_cores, info.num_subcores
    NW = NC * NS
    assert B % (8 * NW) == 0
    DP = 2 * D
    b_per_w = B // NW
    mesh = plsc.VectorSubcoreMesh(core_axis_name="c", subcore_axis_name="s")

    @functools.partial(
        pl.kernel,
        mesh=mesh,
        compiler_params=pltpu.CompilerParams(
            use_tc_tiling_on_sc=True, needs_layout_passes=False
        ),
        out_type=jax.ShapeDtypeStruct((B, DP), jnp.float32),
        scratch_types=[
            pltpu.VMEM((b_per_w,), jnp.int32),
            pltpu.VMEM((b_per_w,), jnp.int32),
            pltpu.VMEM((b_per_w, DP), jnp.float32),
            pltpu.SemaphoreType.DMA,
        ],
    )
    def gather_kernel(idx_hbm, table_hbm, out_hbm, idx_v, bidx_v, rows_v, sem):
        wid = lax.axis_index("s") * NC + lax.axis_index("c")
        base = wid * b_per_w
        pltpu.sync_copy(idx_hbm.at[pl.ds(base, b_per_w)], idx_v)
        for g in range(b_per_w // 16):
            v = idx_v[pl.ds(g * 16, 16)]
            bidx_v[pl.ds(g * 16, 16)] = (
                lax.shift_left(lax.shift_right_logical(v, 10), 9)
                + (v & (_R - 1))
            )
        pltpu.async_copy(table_hbm.at[bidx_v], rows_v, sem).wait()

        def sel_group(g, carry):
            rpos = lax.iota(jnp.int32, 16) + g * 16
            v = idx_v[pl.ds(g * 16, 16)]
            half = (lax.shift_right_logical(v, 9) & 1) * D

            def sel_col(c, carry2):
                vec = plsc.load_gather(rows_v, [rpos, half + c])
                plsc.store_scatter(
                    rows_v, [rpos, jnp.full((16,), 0, jnp.int32) + c], vec
                )
                return carry2

            lax.fori_loop(0, D, sel_col, 0)
            return carry

        lax.fori_loop(0, b_per_w // 16, sel_group, 0)
        pltpu.sync_copy(rows_v, out_hbm.at[pl.ds(base, b_per_w)])

    return gather_kernel


def kernel(beta, emb_weight):
    (B,) = beta.shape
    V, D = emb_weight.shape
    beta = beta.astype(jnp.int32)
    eye = jnp.eye(D, dtype=jnp.float32)
    wt = emb_weight.T
    packed = _make_relayout(V, D)(wt, wt, eye)
    outp = _make_gather(V, D, B, packed.shape[0])(beta, packed)
    return outp[:, :D]


# speedup vs baseline: 3.5303x; 3.5303x over previous
import functools

import jax
import jax.numpy as jnp
from jax import lax
from jax.experimental import pallas as pl
from jax.experimental.pallas import tpu as pltpu
from jax.experimental.pallas import tpu_sc as plsc


@functools.lru_cache(maxsize=None)
def _make_gather(V, D, B):
    info = plsc.get_sparse_core_info()
    NC, NS = info.num_cores, info.num_subcores
    NW = NC * NS
    L = 128
    assert B % (16 * NW) == 0
    b_per_w = B // NW
    n_groups = b_per_w // 16
    mesh = plsc.VectorSubcoreMesh(core_axis_name="c", subcore_axis_name="s")

    @functools.partial(
        pl.kernel,
        mesh=mesh,
        compiler_params=pltpu.CompilerParams(
            use_tc_tiling_on_sc=True, needs_layout_passes=False
        ),
        out_type=jax.ShapeDtypeStruct((B, D), jnp.float32),
        scratch_types=[
            pltpu.VMEM((b_per_w,), jnp.int32),
            pltpu.VMEM((8 * D, L), jnp.float32),
            pltpu.VMEM((16, D), jnp.float32),
            pltpu.SemaphoreType.DMA,
        ],
    )
    def gather_kernel(idx_hbm, table_hbm, out_hbm, idx_v, blocks_v, rows_v,
                      sem):
        wid = lax.axis_index("s") * NC + lax.axis_index("c")
        base = wid * b_per_w
        pltpu.sync_copy(idx_hbm.at[pl.ds(base, b_per_w)], idx_v)

        def group_body(g, carry):
            vec = idx_v[pl.ds(g * 16, 16)]
            for half in range(2):
                for k in range(8):
                    i = vec[half * 8 + k]
                    c = pl.multiple_of(
                        lax.shift_left(lax.shift_right_logical(i, 7), 7), L
                    )
                    pltpu.make_async_copy(
                        table_hbm.at[:, pl.ds(c, L)],
                        blocks_v.at[pl.ds(k * D, D), :],
                        sem,
                    ).start()
                for k in range(8):
                    pltpu.make_async_copy(
                        table_hbm.at[:, pl.ds(0, L)],
                        blocks_v.at[pl.ds(k * D, D), :],
                        sem,
                    ).wait()
                for k in range(8):
                    i = vec[half * 8 + k]
                    lane = jnp.full((16,), i & (L - 1), jnp.int32)
                    r = jnp.full((16,), half * 8 + k, jnp.int32)
                    for q in range(D // 16):
                        rpos = lax.iota(jnp.int32, 16) + (k * D + q * 16)
                        v16 = plsc.load_gather(blocks_v, [rpos, lane])
                        plsc.store_scatter(
                            rows_v,
                            [r, lax.iota(jnp.int32, 16) + q * 16],
                            v16,
                        )
            pltpu.sync_copy(rows_v, out_hbm.at[pl.ds(base + g * 16, 16)])
            return carry

        lax.fori_loop(0, n_groups, group_body, 0)

    return gather_kernel


def kernel(beta, emb_weight):
    (B,) = beta.shape
    V, D = emb_weight.shape
    beta = beta.astype(jnp.int32)
    return _make_gather(V, D, B)(beta, emb_weight.T)


# speedup vs baseline: 4.1315x; 1.1703x over previous
import functools

import jax
import jax.numpy as jnp
from jax import lax
from jax.experimental import pallas as pl
from jax.experimental.pallas import tpu as pltpu
from jax.experimental.pallas import tpu_sc as plsc


@functools.lru_cache(maxsize=None)
def _make_gather(V, D, B):
    info = plsc.get_sparse_core_info()
    NC, NS = info.num_cores, info.num_subcores
    NW = NC * NS
    L = 128
    BURST = 4
    assert B % (16 * NW) == 0
    b_per_w = B // NW
    n_groups = b_per_w // 16
    mesh = plsc.VectorSubcoreMesh(core_axis_name="c", subcore_axis_name="s")

    @functools.partial(
        pl.kernel,
        mesh=mesh,
        compiler_params=pltpu.CompilerParams(
            use_tc_tiling_on_sc=True, needs_layout_passes=False
        ),
        out_type=jax.ShapeDtypeStruct((B, D), jnp.float32),
        scratch_types=[
            pltpu.VMEM((b_per_w,), jnp.int32),
            pltpu.VMEM((3 * BURST * D, L), jnp.float32),
            pltpu.VMEM((16, D), jnp.float32),
            pltpu.SemaphoreType.DMA,
        ],
    )
    def gather_kernel(idx_hbm, table_hbm, out_hbm, idx_v, blocks_v, rows_v,
                      sem):
        wid = lax.axis_index("s") * NC + lax.axis_index("c")
        base = wid * b_per_w
        pltpu.sync_copy(idx_hbm.at[pl.ds(base, b_per_w)], idx_v)

        def fire(vec, h, slot):
            for k in range(BURST):
                i = vec[h * BURST + k]
                c = pl.multiple_of(
                    lax.shift_left(lax.shift_right_logical(i, 7), 7), L
                )
                pltpu.make_async_copy(
                    table_hbm.at[:, pl.ds(c, L)],
                    blocks_v.at[pl.ds((slot * BURST + k) * D, D), :],
                    sem,
                ).start()

        def wait_extract(vec, h, slot):
            for k in range(BURST):
                pltpu.make_async_copy(
                    table_hbm.at[:, pl.ds(0, L)],
                    blocks_v.at[pl.ds((slot * BURST + k) * D, D), :],
                    sem,
                ).wait()
            for k in range(BURST):
                i = vec[h * BURST + k]
                lane = jnp.full((16,), i & (L - 1), jnp.int32)
                r = jnp.full((16,), h * BURST + k, jnp.int32)
                for q in range(D // 16):
                    rpos = lax.iota(jnp.int32, 16) + (
                        (slot * BURST + k) * D + q * 16
                    )
                    v16 = plsc.load_gather(blocks_v, [rpos, lane])
                    plsc.store_scatter(
                        rows_v,
                        [r, lax.iota(jnp.int32, 16) + q * 16],
                        v16,
                    )

        def group_body(g, carry):
            vec = idx_v[pl.ds(g * 16, 16)]
            fire(vec, 0, 0)
            fire(vec, 1, 1)
            fire(vec, 2, 2)
            wait_extract(vec, 0, 0)
            fire(vec, 3, 0)
            wait_extract(vec, 1, 1)
            wait_extract(vec, 2, 2)
            wait_extract(vec, 3, 0)
            pltpu.sync_copy(rows_v, out_hbm.at[pl.ds(base + g * 16, 16)])
            return carry

        lax.fori_loop(0, n_groups, group_body, 0)

    return gather_kernel


def kernel(beta, emb_weight):
    (B,) = beta.shape
    V, D = emb_weight.shape
    beta = beta.astype(jnp.int32)
    return _make_gather(V, D, B)(beta, emb_weight.T)


# speedup vs baseline: 4.4768x; 1.0836x over previous
import functools

import jax
import jax.numpy as jnp
from jax import lax
from jax.experimental import pallas as pl
from jax.experimental.pallas import tpu as pltpu
from jax.experimental.pallas import tpu_sc as plsc


@functools.lru_cache(maxsize=None)
def _make_gather(V, D, B):
    info = plsc.get_sparse_core_info()
    NC, NS = info.num_cores, info.num_subcores
    NW = NC * NS
    L = 128
    BURST = 4
    assert B % (16 * NW) == 0
    b_per_w = B // NW
    n_groups = b_per_w // 16
    mesh = plsc.VectorSubcoreMesh(core_axis_name="c", subcore_axis_name="s")

    @functools.partial(
        pl.kernel,
        mesh=mesh,
        compiler_params=pltpu.CompilerParams(
            use_tc_tiling_on_sc=True, needs_layout_passes=False
        ),
        out_type=jax.ShapeDtypeStruct((B, D), jnp.float32),
        scratch_types=[
            pltpu.VMEM((b_per_w + 16,), jnp.int32),
            pltpu.VMEM((3 * BURST * D, L), jnp.float32),
            pltpu.VMEM((16, D), jnp.float32),
            pltpu.SemaphoreType.DMA,
        ],
    )
    def gather_kernel(idx_hbm, table_hbm, out_hbm, idx_v, blocks_v, rows_v,
                      sem):
        wid = lax.axis_index("s") * NC + lax.axis_index("c")
        base = wid * b_per_w
        pltpu.sync_copy(
            idx_hbm.at[pl.ds(base, b_per_w)], idx_v.at[pl.ds(0, b_per_w)]
        )

        def fire(vec, h, slot):
            for k in range(BURST):
                i = vec[h * BURST + k]
                c = pl.multiple_of(
                    lax.shift_left(lax.shift_right_logical(i, 7), 7), L
                )
                pltpu.make_async_copy(
                    table_hbm.at[:, pl.ds(c, L)],
                    blocks_v.at[pl.ds((slot * BURST + k) * D, D), :],
                    sem,
                ).start()

        def wait_extract(vec, h, slot):
            for k in range(BURST):
                pltpu.make_async_copy(
                    table_hbm.at[:, pl.ds(0, L)],
                    blocks_v.at[pl.ds((slot * BURST + k) * D, D), :],
                    sem,
                ).wait()
            for k in range(BURST):
                i = vec[h * BURST + k]
                lane = jnp.full((16,), i & (L - 1), jnp.int32)
                r = jnp.full((16,), h * BURST + k, jnp.int32)
                for q in range(D // 16):
                    rpos = lax.iota(jnp.int32, 16) + (
                        (slot * BURST + k) * D + q * 16
                    )
                    v16 = plsc.load_gather(blocks_v, [rpos, lane])
                    plsc.store_scatter(
                        rows_v,
                        [r, lax.iota(jnp.int32, 16) + q * 16],
                        v16,
                    )

        vec0 = idx_v[pl.ds(0, 16)]
        fire(vec0, 0, 0)
        fire(vec0, 1, 1)
        fire(vec0, 2, 2)

        def group_body(g, s):
            vec = idx_v[pl.ds(g * 16, 16)]
            vecn = idx_v[pl.ds(g * 16 + 16, 16)]
            s1 = lax.rem(s + 1, 3)
            s2 = lax.rem(s + 2, 3)
            more = g + 1 < n_groups

            wait_extract(vec, 0, s)
            fire(vec, 3, s)
            wait_extract(vec, 1, s1)

            @pl.when(more)
            def _():
                fire(vecn, 0, s1)

            wait_extract(vec, 2, s2)

            @pl.when(more)
            def _():
                fire(vecn, 1, s2)

            wait_extract(vec, 3, s)

            @pl.when(more)
            def _():
                fire(vecn, 2, s)

            pltpu.sync_copy(rows_v, out_hbm.at[pl.ds(base + g * 16, 16)])
            return s1

        lax.fori_loop(0, n_groups, group_body, 0)

    return gather_kernel


def kernel(beta, emb_weight):
    (B,) = beta.shape
    V, D = emb_weight.shape
    beta = beta.astype(jnp.int32)
    return _make_gather(V, D, B)(beta, emb_weight.T)


# speedup vs baseline: 4.6821x; 1.0459x over previous
import functools

import jax
import jax.numpy as jnp
from jax import lax
from jax.experimental import pallas as pl
from jax.experimental.pallas import tpu as pltpu
from jax.experimental.pallas import tpu_sc as plsc


@functools.lru_cache(maxsize=None)
def _make_gather(V, D, B):
    info = plsc.get_sparse_core_info()
    NC, NS = info.num_cores, info.num_subcores
    NW = NC * NS
    L = 128
    BURST = 4
    assert B % (8 * L * NW) == 0 or B % (L * NW) == 0
    b_per_w = B // NW
    n_groups = b_per_w // 16
    mesh = plsc.VectorSubcoreMesh(core_axis_name="c", subcore_axis_name="s")

    @functools.partial(
        pl.kernel,
        mesh=mesh,
        compiler_params=pltpu.CompilerParams(
            use_tc_tiling_on_sc=True, needs_layout_passes=False
        ),
        out_type=jax.ShapeDtypeStruct((D, B), jnp.float32),
        scratch_types=[
            pltpu.VMEM((b_per_w + 16,), jnp.int32),
            pltpu.VMEM((3 * BURST * D, L), jnp.float32),
            pltpu.VMEM((D, L), jnp.float32),
            pltpu.SemaphoreType.DMA,
        ],
    )
    def gather_kernel(idx_hbm, table_hbm, out_hbm, idx_v, blocks_v, rows_t,
                      sem):
        wid = lax.axis_index("s") * NC + lax.axis_index("c")
        base = wid * b_per_w
        pltpu.sync_copy(
            idx_hbm.at[pl.ds(base, b_per_w)], idx_v.at[pl.ds(0, b_per_w)]
        )

        def fire(vec, h, slot):
            for k in range(BURST):
                i = vec[h * BURST + k]
                c = pl.multiple_of(
                    lax.shift_left(lax.shift_right_logical(i, 7), 7), L
                )
                pltpu.make_async_copy(
                    table_hbm.at[:, pl.ds(c, L)],
                    blocks_v.at[pl.ds((slot * BURST + k) * D, D), :],
                    sem,
                ).start()

        def wait_extract(vec, g, h, slot):
            for k in range(BURST):
                pltpu.make_async_copy(
                    table_hbm.at[:, pl.ds(0, L)],
                    blocks_v.at[pl.ds((slot * BURST + k) * D, D), :],
                    sem,
                ).wait()
            rcol_base = lax.rem(g, 8) * 16 + h * BURST
            for k in range(BURST):
                i = vec[h * BURST + k]
                lane = jnp.full((16,), i & (L - 1), jnp.int32)
                rcol = jnp.full((16,), rcol_base + k, jnp.int32)
                for q in range(D // 16):
                    rpos = lax.iota(jnp.int32, 16) + (
                        (slot * BURST + k) * D + q * 16
                    )
                    v16 = plsc.load_gather(blocks_v, [rpos, lane])
                    plsc.store_scatter(
                        rows_t,
                        [lax.iota(jnp.int32, 16) + q * 16, rcol],
                        v16,
                    )

        vec0 = idx_v[pl.ds(0, 16)]
        fire(vec0, 0, 0)
        fire(vec0, 1, 1)
        fire(vec0, 2, 2)

        def group_body(g, s):
            vec = idx_v[pl.ds(g * 16, 16)]
            vecn = idx_v[pl.ds(g * 16 + 16, 16)]
            s1 = lax.rem(s + 1, 3)
            s2 = lax.rem(s + 2, 3)
            more = g + 1 < n_groups

            wait_extract(vec, g, 0, s)
            fire(vec, 3, s)
            wait_extract(vec, g, 1, s1)

            @pl.when(more)
            def _():
                fire(vecn, 0, s1)

            wait_extract(vec, g, 2, s2)

            @pl.when(more)
            def _():
                fire(vecn, 1, s2)

            wait_extract(vec, g, 3, s)

            @pl.when(more)
            def _():
                fire(vecn, 2, s)

            @pl.when(lax.rem(g, 8) == 7)
            def _():
                off = pl.multiple_of(base + (g - 7) * 16, L)
                pltpu.sync_copy(rows_t, out_hbm.at[:, pl.ds(off, L)])

            return s1

        lax.fori_loop(0, n_groups, group_body, 0)

    return gather_kernel


def kernel(beta, emb_weight):
    (B,) = beta.shape
    V, D = emb_weight.shape
    beta = beta.astype(jnp.int32)
    out_t = _make_gather(V, D, B)(beta, emb_weight.T)
    return out_t.T


# speedup vs baseline: 4.6863x; 1.0009x over previous
import functools

import jax
import jax.numpy as jnp
from jax import lax
from jax.experimental import pallas as pl
from jax.experimental.pallas import tpu as pltpu
from jax.experimental.pallas import tpu_sc as plsc


@functools.lru_cache(maxsize=None)
def _make_gather(V, D, B):
    info = plsc.get_sparse_core_info()
    NC, NS = info.num_cores, info.num_subcores
    NW = NC * NS
    L = 128
    BURST = 4
    assert B % (8 * L * NW) == 0 or B % (L * NW) == 0
    b_per_w = B // NW
    n_groups = b_per_w // 16
    mesh = plsc.VectorSubcoreMesh(core_axis_name="c", subcore_axis_name="s")

    @functools.partial(
        pl.kernel,
        mesh=mesh,
        compiler_params=pltpu.CompilerParams(
            use_tc_tiling_on_sc=True, needs_layout_passes=False
        ),
        out_type=jax.ShapeDtypeStruct((D, B), jnp.float32),
        scratch_types=[
            pltpu.VMEM((b_per_w + 16,), jnp.int32),
            pltpu.VMEM((3 * BURST * D, L), jnp.float32),
            pltpu.VMEM((2 * D, L), jnp.float32),
            pltpu.SemaphoreType.DMA,
            pltpu.SemaphoreType.DMA,
        ],
    )
    def gather_kernel(idx_hbm, table_hbm, out_hbm, idx_v, blocks_v, rows_t,
                      sem, osem):
        wid = lax.axis_index("s") * NC + lax.axis_index("c")
        base = wid * b_per_w
        pltpu.sync_copy(
            idx_hbm.at[pl.ds(base, b_per_w)], idx_v.at[pl.ds(0, b_per_w)]
        )

        def fire(vec, h, slot):
            for k in range(BURST):
                i = vec[h * BURST + k]
                c = pl.multiple_of(
                    lax.shift_left(lax.shift_right_logical(i, 7), 7), L
                )
                pltpu.make_async_copy(
                    table_hbm.at[:, pl.ds(c, L)],
                    blocks_v.at[pl.ds((slot * BURST + k) * D, D), :],
                    sem,
                ).start()

        def wait_extract(vec, g, h, slot):
            for k in range(BURST):
                pltpu.make_async_copy(
                    table_hbm.at[:, pl.ds(0, L)],
                    blocks_v.at[pl.ds((slot * BURST + k) * D, D), :],
                    sem,
                ).wait()
            rcol_base = lax.rem(g, 8) * 16 + h * BURST
            par = lax.rem(lax.div(g, 8), 2) * D
            for k in range(BURST):
                i = vec[h * BURST + k]
                lane = jnp.full((16,), i & (L - 1), jnp.int32)
                rcol = jnp.full((16,), rcol_base + k, jnp.int32)
                for q in range(D // 16):
                    rpos = lax.iota(jnp.int32, 16) + (
                        (slot * BURST + k) * D + q * 16
                    )
                    v16 = plsc.load_gather(blocks_v, [rpos, lane])
                    plsc.store_scatter(
                        rows_t,
                        [lax.iota(jnp.int32, 16) + q * 16 + par, rcol],
                        v16,
                    )

        vec0 = idx_v[pl.ds(0, 16)]
        fire(vec0, 0, 0)
        fire(vec0, 1, 1)
        fire(vec0, 2, 2)

        def group_body(g, s):
            vec = idx_v[pl.ds(g * 16, 16)]
            vecn = idx_v[pl.ds(g * 16 + 16, 16)]
            s1 = lax.rem(s + 1, 3)
            s2 = lax.rem(s + 2, 3)
            more = g + 1 < n_groups

            @pl.when((lax.rem(g, 8) == 0) & (g >= 16))
            def _():
                pltpu.make_async_copy(
                    rows_t.at[pl.ds(0, D), :],
                    out_hbm.at[:, pl.ds(base, L)],
                    osem,
                ).wait()

            wait_extract(vec, g, 0, s)
            fire(vec, 3, s)
            wait_extract(vec, g, 1, s1)

            @pl.when(more)
            def _():
                fire(vecn, 0, s1)

            wait_extract(vec, g, 2, s2)

            @pl.when(more)
            def _():
                fire(vecn, 1, s2)

            wait_extract(vec, g, 3, s)

            @pl.when(more)
            def _():
                fire(vecn, 2, s)

            @pl.when(lax.rem(g, 8) == 7)
            def _():
                par = lax.rem(lax.div(g, 8), 2) * D
                off = pl.multiple_of(base + (g - 7) * 16, L)
                pltpu.make_async_copy(
                    rows_t.at[pl.ds(par, D), :],
                    out_hbm.at[:, pl.ds(off, L)],
                    osem,
                ).start()

            return s1

        lax.fori_loop(0, n_groups, group_body, 0)
        for _ in range(2):
            pltpu.make_async_copy(
                rows_t.at[pl.ds(0, D), :],
                out_hbm.at[:, pl.ds(base, L)],
                osem,
            ).wait()

    return gather_kernel


def kernel(beta, emb_weight):
    (B,) = beta.shape
    V, D = emb_weight.shape
    beta = beta.astype(jnp.int32)
    out_t = _make_gather(V, D, B)(beta, emb_weight.T)
    return out_t.T
